# baseline (speedup 1.0000x reference)
import functools

import jax
import jax.numpy as jnp
from jax import lax
from jax.experimental import pallas as pl
from jax.experimental.pallas import tpu as pltpu
from jax.experimental.pallas import tpu_sc as plsc

_B = 512
_C = 2048
_N = 4096
_NBITS = 14
_M = 1 << _NBITS

_BN = 512

_NW = 32
_TOT = _B * _N
_PW = _TOT // _NW
_CH = 8192
_NCH = _PW // _CH
_SUB = 128
_NSUB = _CH // _SUB


def _addr_body(x_ref, wlo_ref, whi_ref, out_ref):
    dims = (((1,), (1,)), ((), ()))
    lo = lax.dot_general(x_ref[...], wlo_ref[...], dims,
                         preferred_element_type=jnp.float32)
    hi = lax.dot_general(x_ref[...], whi_ref[...], dims,
                         preferred_element_type=jnp.float32)
    n0 = pl.program_id(0) * _BN
    col = lax.broadcasted_iota(jnp.int32, lo.shape, 1) + n0
    out_ref[...] = (lo.astype(jnp.int32) + (hi.astype(jnp.int32) << 7)
                    + col * _M)


_addr_call = pl.pallas_call(
    _addr_body,
    grid=(_N // _BN,),
    in_specs=[
        pl.BlockSpec((_B, _C), lambda j: (0, 0)),
        pl.BlockSpec((_BN, _C), lambda j: (j, 0)),
        pl.BlockSpec((_BN, _C), lambda j: (j, 0)),
    ],
    out_specs=pl.BlockSpec((_B, _BN), lambda j: (0, j)),
    out_shape=jax.ShapeDtypeStruct((_B, _N), jnp.int32),
)


def _lookup_body(idx_hbm, mem_hbm, out_hbm, idx_v, val_v, sem):
    wid = lax.axis_index("s") * 2 + lax.axis_index("c")
    base = wid * _PW

    def outer(i, carry):
        o0 = base + i * _CH
        pltpu.sync_copy(idx_hbm.at[pl.ds(o0, _CH)], idx_v)
        copies = [
            pltpu.async_copy(
                mem_hbm.at[idx_v.at[pl.ds(k * _SUB, _SUB)]],
                val_v.at[pl.ds(k * _SUB, _SUB)],
                sem,
            )
            for k in range(_NSUB)
        ]
        for c in copies:
            c.wait()

        def inner(j, c):
            v = val_v[pl.ds(j * 16, 16)]
            one = jnp.full((16,), 1, jnp.int32)
            zero = jnp.zeros((16,), jnp.int32)
            val_v[pl.ds(j * 16, 16)] = jnp.where(v == one, one, zero)
            return c

        lax.fori_loop(0, _CH // 16, inner, 0, unroll=8)
        pltpu.sync_copy(val_v, out_hbm.at[pl.ds(o0, _CH)])
        return carry

    lax.fori_loop(0, _NCH, outer, 0)


_lookup_call = functools.partial(
    pl.kernel,
    mesh=plsc.VectorSubcoreMesh(core_axis_name="c", subcore_axis_name="s"),
    out_type=jax.ShapeDtypeStruct((_TOT,), jnp.int32),
    scratch_types=[
        pltpu.VMEM((_CH,), jnp.int32),
        pltpu.VMEM((_CH,), jnp.int32),
        pltpu.SemaphoreType.DMA,
    ],
)(_lookup_body)


def kernel(input_bits, connections, memory):
    x = input_bits.astype(jnp.bfloat16)
    shifts = (1 << jnp.arange(_NBITS, dtype=jnp.int32)).astype(jnp.float32)
    c_iota = jnp.arange(_C, dtype=jnp.int32)
    onehot = (connections[:, :, None] == c_iota[None, None, :]).astype(
        jnp.float32)
    wlo = jnp.sum(onehot[:, :7, :] * shifts[None, :7, None],
                  axis=1).astype(jnp.bfloat16)
    whi = jnp.sum(onehot[:, 7:, :] * (shifts[None, 7:, None] / 128.0),
                  axis=1).astype(jnp.bfloat16)

    flat_idx = _addr_call(x, wlo, whi).reshape(_TOT)
    vals = _lookup_call(flat_idx, memory.reshape(_N * _M))
    return vals.reshape(_B, _N).astype(jnp.bool_)


# speedup vs baseline: 1.4190x; 1.4190x over previous
import functools

import jax
import jax.numpy as jnp
from jax import lax
from jax.experimental import pallas as pl
from jax.experimental.pallas import tpu as pltpu
from jax.experimental.pallas import tpu_sc as plsc

_B = 512
_C = 2048
_N = 4096
_NBITS = 14
_M = 1 << _NBITS

_BN = 512

_NW = 32
_TOT = _B * _N
_PW = _TOT // _NW
_CH = 8192
_NCH = _PW // _CH
_SUB = 128
_NSUB = _CH // _SUB


def _addr_body(x_ref, wlo_ref, whi_ref, out_ref):
    dims = (((1,), (1,)), ((), ()))
    lo = lax.dot_general(x_ref[...], wlo_ref[...], dims,
                         preferred_element_type=jnp.float32)
    hi = lax.dot_general(x_ref[...], whi_ref[...], dims,
                         preferred_element_type=jnp.float32)
    n0 = pl.program_id(0) * _BN
    n = lax.broadcasted_iota(jnp.int32, lo.shape, 1) + n0
    a = lo.astype(jnp.int32) + (hi.astype(jnp.int32) << 7)
    out_ref[...] = (((n >> 3) << 17) + ((a >> 7) << 10)
                    + ((n & 7) << 7) + (a & 127))


_addr_call = pl.pallas_call(
    _addr_body,
    grid=(_N // _BN,),
    in_specs=[
        pl.BlockSpec((_B, _C), lambda j: (0, 0)),
        pl.BlockSpec((_BN, _C), lambda j: (j, 0)),
        pl.BlockSpec((_BN, _C), lambda j: (j, 0)),
    ],
    out_specs=pl.BlockSpec((_B, _BN), lambda j: (0, j)),
    out_shape=jax.ShapeDtypeStruct((_B, _N), jnp.int32),
)


def _lookup_body(idx_hbm, mem_hbm, out_hbm, idx_v, val_v, sem):
    wid = lax.axis_index("s") * 2 + lax.axis_index("c")
    base = wid * _PW

    def outer(i, carry):
        o0 = base + i * _CH
        pltpu.sync_copy(idx_hbm.at[pl.ds(o0, _CH)], idx_v)
        copies = [
            pltpu.async_copy(
                mem_hbm.at[idx_v.at[pl.ds(k * _SUB, _SUB)]],
                val_v.at[pl.ds(k * _SUB, _SUB)],
                sem,
            )
            for k in range(_NSUB)
        ]
        for c in copies:
            c.wait()

        def inner(j, c):
            v = val_v[pl.ds(j * 16, 16)]
            one = jnp.full((16,), 1, jnp.int32)
            zero = jnp.zeros((16,), jnp.int32)
            val_v[pl.ds(j * 16, 16)] = jnp.where(v == one, one, zero)
            return c

        lax.fori_loop(0, _CH // 16, inner, 0, unroll=8)
        pltpu.sync_copy(val_v, out_hbm.at[pl.ds(o0, _CH)])
        return carry

    lax.fori_loop(0, _NCH, outer, 0)


_lookup_call = functools.partial(
    pl.kernel,
    mesh=plsc.VectorSubcoreMesh(core_axis_name="c", subcore_axis_name="s"),
    out_type=jax.ShapeDtypeStruct((_TOT,), jnp.int32),
    scratch_types=[
        pltpu.VMEM((_CH,), jnp.int32),
        pltpu.VMEM((_CH,), jnp.int32),
        pltpu.SemaphoreType.DMA,
    ],
)(_lookup_body)


def kernel(input_bits, connections, memory):
    x = input_bits.astype(jnp.bfloat16)
    shifts = (1 << jnp.arange(_NBITS, dtype=jnp.int32)).astype(jnp.float32)
    c_iota = jnp.arange(_C, dtype=jnp.int32)
    onehot = (connections[:, :, None] == c_iota[None, None, :]).astype(
        jnp.float32)
    wlo = jnp.sum(onehot[:, :7, :] * shifts[None, :7, None],
                  axis=1).astype(jnp.bfloat16)
    whi = jnp.sum(onehot[:, 7:, :] * (shifts[None, 7:, None] / 128.0),
                  axis=1).astype(jnp.bfloat16)

    idx = _addr_call(x, wlo, whi)
    idx_flat = (idx.reshape(_B // 8, 8, _N // 128, 128)
                .transpose(0, 2, 1, 3).reshape(_TOT))
    mem_flat = (memory.reshape(_N // 8, 8, _M // 128, 128)
                .transpose(0, 2, 1, 3).reshape(_N * _M))
    vals = _lookup_call(idx_flat, mem_flat)
    out = (vals.reshape(_B // 8, _N // 128, 8, 128)
           .transpose(0, 2, 1, 3).reshape(_B, _N))
    return out.astype(jnp.bool_)


# speedup vs baseline: 2.1731x; 1.5314x over previous
import functools

import jax
import jax.numpy as jnp
from jax import lax
from jax.experimental import pallas as pl
from jax.experimental.pallas import tpu as pltpu
from jax.experimental.pallas import tpu_sc as plsc

_B = 512
_C = 2048
_N = 4096
_NBITS = 14
_M = 1 << _NBITS

_BN = 512

_NW = 32
_TOT = _B * _N
_PW = _TOT // _NW
_CH = 8192
_NCH = _PW // _CH
_SUB = 128
_NSUB = _CH // _SUB


def _addr_body(x_ref, w_ref, out_ref):
    w = w_ref[...]
    whi = jnp.floor(w * (1.0 / 128.0))
    wlo = w - whi * 128.0
    dims = (((1,), (1,)), ((), ()))
    lo = lax.dot_general(x_ref[...], wlo.astype(jnp.bfloat16), dims,
                         preferred_element_type=jnp.float32)
    hi = lax.dot_general(x_ref[...], whi.astype(jnp.bfloat16), dims,
                         preferred_element_type=jnp.float32)
    n0 = pl.program_id(0) * _BN
    n = lax.broadcasted_iota(jnp.int32, lo.shape, 1) + n0
    a = lo.astype(jnp.int32) + (hi.astype(jnp.int32) << 7)
    out_ref[...] = (((n >> 3) << 17) + ((a >> 7) << 10)
                    + ((n & 7) << 7) + (a & 127))


_addr_call = pl.pallas_call(
    _addr_body,
    grid=(_N // _BN,),
    in_specs=[
        pl.BlockSpec((_B, _C), lambda j: (0, 0)),
        pl.BlockSpec((_BN, _C), lambda j: (j, 0)),
    ],
    out_specs=pl.BlockSpec((_B, _BN), lambda j: (0, j)),
    out_shape=jax.ShapeDtypeStruct((_B, _N), jnp.int32),
)


def _lookup_body(idx_hbm, mem_hbm, out_hbm, idx_v, val_v, sem):
    wid = lax.axis_index("s") * 2 + lax.axis_index("c")
    base = wid * _PW

    def outer(i, carry):
        o0 = base + i * _CH
        pltpu.sync_copy(idx_hbm.at[pl.ds(o0, _CH)], idx_v)
        copies = [
            pltpu.async_copy(
                mem_hbm.at[idx_v.at[pl.ds(k * _SUB, _SUB)]],
                val_v.at[pl.ds(k * _SUB, _SUB)],
                sem,
            )
            for k in range(_NSUB)
        ]
        for c in copies:
            c.wait()

        def inner(j, c):
            v = val_v[pl.ds(j * 16, 16)]
            one = jnp.full((16,), 1, jnp.int32)
            zero = jnp.zeros((16,), jnp.int32)
            val_v[pl.ds(j * 16, 16)] = jnp.where(v == one, one, zero)
            return c

        lax.fori_loop(0, _CH // 16, inner, 0, unroll=8)
        pltpu.sync_copy(val_v, out_hbm.at[pl.ds(o0, _CH)])
        return carry

    lax.fori_loop(0, _NCH, outer, 0)


_lookup_call = functools.partial(
    pl.kernel,
    mesh=plsc.VectorSubcoreMesh(core_axis_name="c", subcore_axis_name="s"),
    out_type=jax.ShapeDtypeStruct((_TOT,), jnp.int32),
    scratch_types=[
        pltpu.VMEM((_CH,), jnp.int32),
        pltpu.VMEM((_CH,), jnp.int32),
        pltpu.SemaphoreType.DMA,
    ],
)(_lookup_body)


def kernel(input_bits, connections, memory):
    x = input_bits.astype(jnp.bfloat16)
    n_rows = jnp.broadcast_to(jnp.arange(_N, dtype=jnp.int32)[:, None],
                              (_N, _NBITS))
    pw = jnp.broadcast_to(
        (1 << jnp.arange(_NBITS)).astype(jnp.float32)[None, :], (_N, _NBITS))
    w = jnp.zeros((_N, _C), jnp.float32).at[n_rows, connections].add(pw)

    idx = _addr_call(x, w)
    idx_flat = (idx.reshape(_B // 8, 8, _N // 128, 128)
                .transpose(0, 2, 1, 3).reshape(_TOT))
    mem_flat = (memory.reshape(_N // 8, 8, _M // 128, 128)
                .transpose(0, 2, 1, 3).reshape(_N * _M))
    vals = _lookup_call(idx_flat, mem_flat)
    out = (vals.reshape(_B // 8, _N // 128, 8, 128)
           .transpose(0, 2, 1, 3).reshape(_B, _N))
    return out.astype(jnp.bool_)


# speedup vs baseline: 2.1776x; 1.0020x over previous
import functools

import jax
import jax.numpy as jnp
from jax import lax
from jax.experimental import pallas as pl
from jax.experimental.pallas import tpu as pltpu
from jax.experimental.pallas import tpu_sc as plsc

_B = 512
_C = 2048
_N = 4096
_NBITS = 14
_M = 1 << _NBITS

_BN = 512

_NW = 32
_TOT = _B * _N
_PW = _TOT // _NW
_CH = 8192
_NCH = _PW // _CH
_SUB = 128
_NSUB = _CH // _SUB


def _addr_body(x_ref, w_ref, out_ref):
    w = w_ref[...]
    whi = jnp.floor(w * (1.0 / 128.0))
    wlo = w - whi * 128.0
    dims = (((1,), (1,)), ((), ()))
    lo = lax.dot_general(x_ref[...], wlo.astype(jnp.bfloat16), dims,
                         preferred_element_type=jnp.float32)
    hi = lax.dot_general(x_ref[...], whi.astype(jnp.bfloat16), dims,
                         preferred_element_type=jnp.float32)
    n0 = pl.program_id(0) * _BN
    n = lax.broadcasted_iota(jnp.int32, lo.shape, 1) + n0
    a = lo.astype(jnp.int32) + (hi.astype(jnp.int32) << 7)
    out_ref[...] = (((n >> 3) << 17) + ((a >> 7) << 10)
                    + ((n & 7) << 7) + (a & 127))


_addr_call = pl.pallas_call(
    _addr_body,
    grid=(_N // _BN,),
    in_specs=[
        pl.BlockSpec((_B, _C), lambda j: (0, 0)),
        pl.BlockSpec((_BN, _C), lambda j: (j, 0)),
    ],
    out_specs=pl.BlockSpec((_B, _BN), lambda j: (0, j)),
    out_shape=jax.ShapeDtypeStruct((_B, _N), jnp.int32),
)


def _lookup_body(idx_hbm, mem_hbm, out_hbm, idx_v, val_v, sem):
    wid = lax.axis_index("s") * 2 + lax.axis_index("c")
    base = wid * _PW

    def outer(i, carry):
        o0 = base + i * _CH
        pltpu.sync_copy(idx_hbm.at[pl.ds(o0, _CH)], idx_v)
        copies = [
            pltpu.async_copy(
                mem_hbm.at[idx_v.at[pl.ds(k * _SUB, _SUB)]],
                val_v.at[pl.ds(k * _SUB, _SUB)],
                sem,
            )
            for k in range(_NSUB)
        ]
        for c in copies:
            c.wait()

        def inner(j, c):
            v = val_v[pl.ds(j * 16, 16)]
            one = jnp.full((16,), 1, jnp.int32)
            zero = jnp.zeros((16,), jnp.int32)
            val_v[pl.ds(j * 16, 16)] = jnp.where(v == one, one, zero)
            return c

        lax.fori_loop(0, _CH // 16, inner, 0, unroll=8)
        pltpu.sync_copy(val_v, out_hbm.at[pl.ds(o0, _CH)])
        return carry

    lax.fori_loop(0, _NCH, outer, 0)


_lookup_call = functools.partial(
    pl.kernel,
    mesh=plsc.VectorSubcoreMesh(core_axis_name="c", subcore_axis_name="s"),
    out_type=jax.ShapeDtypeStruct((_TOT,), jnp.int32),
    scratch_types=[
        pltpu.VMEM((_CH,), jnp.int32),
        pltpu.VMEM((_CH,), jnp.int32),
        pltpu.SemaphoreType.DMA,
    ],
)(_lookup_body)


def kernel(input_bits, connections, memory):
    x = input_bits.astype(jnp.bfloat16)
    flat_c = (jnp.arange(_N, dtype=jnp.int32)[:, None] * _C
              + connections).reshape(_N * _NBITS)
    pw = jnp.broadcast_to(
        (1 << jnp.arange(_NBITS)).astype(jnp.float32)[None, :],
        (_N, _NBITS)).reshape(_N * _NBITS)
    w = (jnp.zeros((_N * _C,), jnp.float32).at[flat_c].add(pw)
         .reshape(_N, _C))

    idx = _addr_call(x, w)
    idx_flat = (idx.reshape(_B // 8, 8, _N // 128, 128)
                .transpose(0, 2, 1, 3).reshape(_TOT))
    mem_flat = (memory.reshape(_N // 8, 8, _M // 128, 128)
                .transpose(0, 2, 1, 3).reshape(_N * _M))
    vals = _lookup_call(idx_flat, mem_flat)
    out = (vals.reshape(_B // 8, _N // 128, 8, 128)
           .transpose(0, 2, 1, 3).reshape(_B, _N))
    return out.astype(jnp.bool_)
